# baseline (device time: 40644 ns/iter reference)
import jax
import jax.numpy as jnp
from jax import lax
from jax.experimental import pallas as pl
from jax.experimental.pallas import tpu as pltpu

N_DEV = 16
B = 256
D = 256
BLK = B // N_DEV


def kernel(x, Win0, Wout0, Win1, Wout1, Win2, Wout2):
    def body(x_ref, win0_ref, wout0_ref, win1_ref, wout1_ref, win2_ref,
             wout2_ref, out_ref, p_ref, rs_buf, x_buf,
             send_sems, rs_sems, ag_sems):
        my = lax.axis_index("i")

        barrier = pltpu.get_barrier_semaphore()
        for k in range(1, N_DEV):
            pl.semaphore_signal(
                barrier, inc=1,
                device_id=((my + k) % N_DEV,),
                device_id_type=pl.DeviceIdType.MESH,
            )
        pl.semaphore_wait(barrier, N_DEV - 1)

        wins = [win0_ref, win1_ref, win2_ref]
        wouts = [wout0_ref, wout1_ref, wout2_ref]

        xv = x_ref[...]
        for layer in range(3):
            w_in = wins[layer][...].astype(jnp.bfloat16)
            w_out = wouts[layer][...].astype(jnp.bfloat16)
            h = jnp.dot(xv.astype(jnp.bfloat16), w_in,
                        preferred_element_type=jnp.float32)
            h = jnp.maximum(h, 0.0)
            part = jnp.dot(h.astype(jnp.bfloat16), w_out,
                           preferred_element_type=jnp.float32)
            p_ref[...] = part

            rs = []
            for k in range(1, N_DEV):
                dst = (my + k) % N_DEV
                r = pltpu.make_async_remote_copy(
                    src_ref=p_ref.at[pl.ds(dst * BLK, BLK), :],
                    dst_ref=rs_buf.at[k],
                    send_sem=send_sems.at[k],
                    recv_sem=rs_sems.at[k],
                    device_id=(dst,),
                    device_id_type=pl.DeviceIdType.MESH,
                )
                r.start()
                rs.append(r)
            rs_buf[0, :, :] = p_ref[pl.ds(my * BLK, BLK), :]
            for r in rs:
                r.wait_recv()
            for r in rs:
                r.wait_send()
            acc = jnp.sum(rs_buf[...], axis=0)

            if layer == 2:
                out_ref[...] = acc
            else:
                x_buf[pl.ds(my * BLK, BLK), :] = acc
                ag = []
                for k in range(1, N_DEV):
                    dst = (my + k) % N_DEV
                    r = pltpu.make_async_remote_copy(
                        src_ref=x_buf.at[pl.ds(my * BLK, BLK), :],
                        dst_ref=x_buf.at[pl.ds(my * BLK, BLK), :],
                        send_sem=send_sems.at[k],
                        recv_sem=ag_sems.at[k],
                        device_id=(dst,),
                        device_id_type=pl.DeviceIdType.MESH,
                    )
                    r.start()
                    ag.append(r)
                for r in ag:
                    r.wait_recv()
                for r in ag:
                    r.wait_send()
                xv = x_buf[...]

    return pl.pallas_call(
        body,
        out_shape=jax.ShapeDtypeStruct((BLK, D), jnp.float32),
        in_specs=[pl.BlockSpec(memory_space=pltpu.VMEM)] * 7,
        out_specs=pl.BlockSpec(memory_space=pltpu.VMEM),
        scratch_shapes=[
            pltpu.VMEM((B, D), jnp.float32),
            pltpu.VMEM((N_DEV, BLK, D), jnp.float32),
            pltpu.VMEM((B, D), jnp.float32),
            pltpu.SemaphoreType.DMA((N_DEV,)),
            pltpu.SemaphoreType.DMA((N_DEV,)),
            pltpu.SemaphoreType.DMA((N_DEV,)),
        ],
        compiler_params=pltpu.CompilerParams(collective_id=0),
    )(x, Win0, Wout0, Win1, Wout1, Win2, Wout2)


# device time: 36098 ns/iter; 1.1259x vs baseline; 1.1259x over previous
import jax
import jax.numpy as jnp
from jax import lax
from jax.experimental import pallas as pl
from jax.experimental.pallas import tpu as pltpu

N_DEV = 16
B = 256
D = 256
BLK = B // N_DEV


def kernel(x, Win0, Wout0, Win1, Wout1, Win2, Wout2):
    def body(x_ref, win0_ref, wout0_ref, win1_ref, wout1_ref, win2_ref,
             wout2_ref, out_ref, p_ref, rs_buf, x_buf,
             send_sems, rs_sems, ag_sems):
        my = lax.axis_index("i")

        barrier = pltpu.get_barrier_semaphore()
        for k in range(1, N_DEV):
            pl.semaphore_signal(
                barrier, inc=1,
                device_id=((my + k) % N_DEV,),
                device_id_type=pl.DeviceIdType.MESH,
            )
        pl.semaphore_wait(barrier, N_DEV - 1)

        wins = [win0_ref, win1_ref, win2_ref]
        wouts = [wout0_ref, wout1_ref, wout2_ref]

        xv = x_ref[...]
        for layer in range(3):
            w_in = wins[layer][...].astype(jnp.bfloat16)
            w_out = wouts[layer][...].astype(jnp.bfloat16)
            h = jnp.dot(xv.astype(jnp.bfloat16), w_in,
                        preferred_element_type=jnp.float32)
            h = jnp.maximum(h, 0.0)
            part = jnp.dot(h.astype(jnp.bfloat16), w_out,
                           preferred_element_type=jnp.float32)
            p_ref[...] = part.astype(jnp.bfloat16)

            rs = []
            for k in range(1, N_DEV):
                dst = (my + k) % N_DEV
                r = pltpu.make_async_remote_copy(
                    src_ref=p_ref.at[pl.ds(dst * BLK, BLK), :],
                    dst_ref=rs_buf.at[k],
                    send_sem=send_sems.at[k],
                    recv_sem=rs_sems.at[k],
                    device_id=(dst,),
                    device_id_type=pl.DeviceIdType.MESH,
                )
                r.start()
                rs.append(r)
            rs_buf[0, :, :] = p_ref[pl.ds(my * BLK, BLK), :]
            for r in rs:
                r.wait_recv()
            for r in rs:
                r.wait_send()
            acc = jnp.sum(rs_buf[...].astype(jnp.float32), axis=0)

            if layer == 2:
                out_ref[...] = acc
            else:
                x_buf[pl.ds(my * BLK, BLK), :] = acc.astype(jnp.bfloat16)
                ag = []
                for k in range(1, N_DEV):
                    dst = (my + k) % N_DEV
                    r = pltpu.make_async_remote_copy(
                        src_ref=x_buf.at[pl.ds(my * BLK, BLK), :],
                        dst_ref=x_buf.at[pl.ds(my * BLK, BLK), :],
                        send_sem=send_sems.at[k],
                        recv_sem=ag_sems.at[k],
                        device_id=(dst,),
                        device_id_type=pl.DeviceIdType.MESH,
                    )
                    r.start()
                    ag.append(r)
                for r in ag:
                    r.wait_recv()
                for r in ag:
                    r.wait_send()
                xv = x_buf[...]

    return pl.pallas_call(
        body,
        out_shape=jax.ShapeDtypeStruct((BLK, D), jnp.float32),
        in_specs=[pl.BlockSpec(memory_space=pltpu.VMEM)] * 7,
        out_specs=pl.BlockSpec(memory_space=pltpu.VMEM),
        scratch_shapes=[
            pltpu.VMEM((B, D), jnp.bfloat16),
            pltpu.VMEM((N_DEV, BLK, D), jnp.bfloat16),
            pltpu.VMEM((B, D), jnp.bfloat16),
            pltpu.SemaphoreType.DMA((N_DEV,)),
            pltpu.SemaphoreType.DMA((N_DEV,)),
            pltpu.SemaphoreType.DMA((N_DEV,)),
        ],
        compiler_params=pltpu.CompilerParams(collective_id=0),
    )(x, Win0, Wout0, Win1, Wout1, Win2, Wout2)
